# baseline (speedup 1.0000x reference)
import jax
import jax.numpy as jnp
from jax.experimental import pallas as pl
from jax.experimental.pallas import tpu as pltpu

_K = 4096
_NCHUNK = 4
_CK = _K // _NCHUNK


def _dotT(a, b):
    return jax.lax.dot_general(
        a, b, (((1,), (1,)), ((), ())), preferred_element_type=jnp.float32
    )


def _gate_body(z_ref, tf_ref, tr_ref, w1_ref, b1_ref, a_ref,
               lnw_ref, lnb_ref, w2_ref, b2_ref, out_ref,
               buf0, buf1, tail_buf, sem0, sem1, sem_t):
    bufs = (buf0, buf1)
    sems = (sem0, sem1)

    def chunk_copy(i, buf, sem):
        return pltpu.make_async_copy(
            w1_ref.at[:, pl.ds(i * _CK, _CK)], buf, sem)

    tail_cp = pltpu.make_async_copy(
        w1_ref.at[:, pl.ds(_K, 2)], tail_buf, sem_t)
    tail_cp.start()
    chunk_copy(0, buf0, sem0).start()
    chunk_copy(1, buf1, sem1).start()

    h = jnp.zeros((1, 128), jnp.float32)
    for i in range(_NCHUNK):
        cp = chunk_copy(i, bufs[i % 2], sems[i % 2])
        cp.wait()
        h = h + _dotT(z_ref[:, pl.ds(i * _CK, _CK)], bufs[i % 2][...])
        if i + 2 < _NCHUNK:
            chunk_copy(i + 2, bufs[i % 2], sems[i % 2]).start()

    tail_cp.wait()
    h = h + _dotT(tf_ref[...], tail_buf[:, 0:1])
    h = h + _dotT(tr_ref[...], tail_buf[:, 1:2])
    h = h + b1_ref[...]
    h = jnp.maximum(h, 0.0) + a_ref[...] * jnp.minimum(h, 0.0)
    mu = jnp.mean(h, axis=1, keepdims=True)
    d = h - mu
    var = jnp.mean(d * d, axis=1, keepdims=True)
    hn = d * jax.lax.rsqrt(var + 1e-5) * lnw_ref[...] + lnb_ref[...]
    logits = _dotT(hn, w2_ref[...]) + b2_ref[...]
    m = jnp.max(logits, axis=1, keepdims=True)
    e = jnp.exp(logits - m)
    s = jnp.sum(e, axis=1, keepdims=True)
    out_ref[...] = e / s


@jax.jit
def _gate(z, tf, tr, W1, b1, a, lnw, lnb, W2, b2):
    vmem = pl.BlockSpec(memory_space=pltpu.VMEM)
    return pl.pallas_call(
        _gate_body,
        out_shape=jax.ShapeDtypeStruct((1, 64), jnp.float32),
        in_specs=[vmem, vmem, vmem,
                  pl.BlockSpec(memory_space=pltpu.MemorySpace.HBM),
                  vmem, vmem, vmem, vmem, vmem, vmem],
        out_specs=vmem,
        scratch_shapes=[
            pltpu.VMEM((128, _CK), jnp.float32),
            pltpu.VMEM((128, _CK), jnp.float32),
            pltpu.VMEM((128, 2), jnp.float32),
            pltpu.SemaphoreType.DMA,
            pltpu.SemaphoreType.DMA,
            pltpu.SemaphoreType.DMA,
        ],
    )(z, tf, tr, W1, b1, a, lnw, lnb, W2, b2)


def kernel(z_refined, trust_form, trust_role, W1, b1, prelu_a, ln_w, ln_b, W2, b2):
    return _gate(
        z_refined,
        trust_form.reshape(1, 1),
        trust_role.reshape(1, 1),
        W1,
        b1.reshape(1, 128),
        prelu_a.reshape(1, 1),
        ln_w.reshape(1, 128),
        ln_b.reshape(1, 128),
        W2,
        b2.reshape(1, 64),
    )


# speedup vs baseline: 1.2211x; 1.2211x over previous
import jax
import jax.numpy as jnp
from jax.experimental import pallas as pl
from jax.experimental.pallas import tpu as pltpu


def _dotT(a, b):
    return jax.lax.dot_general(
        a, b, (((1,), (1,)), ((), ())), preferred_element_type=jnp.float32
    )


def _gate_body(z_hbm, tf_hbm, tr_hbm, w1_hbm, b1_hbm, a_hbm,
               lnw_hbm, lnb_hbm, w2_hbm, b2_hbm, out_ref,
               w1_v, z_v, tf_v, tr_v, b1_v, a_v, lnw_v, lnb_v, w2_v, b2_v,
               sem_w1, sem_small):
    cp_w1 = pltpu.make_async_copy(w1_hbm, w1_v, sem_w1)
    cp_w1.start()
    small = [
        pltpu.make_async_copy(z_hbm, z_v, sem_small),
        pltpu.make_async_copy(tf_hbm, tf_v, sem_small),
        pltpu.make_async_copy(tr_hbm, tr_v, sem_small),
        pltpu.make_async_copy(b1_hbm, b1_v, sem_small),
        pltpu.make_async_copy(a_hbm, a_v, sem_small),
        pltpu.make_async_copy(lnw_hbm, lnw_v, sem_small),
        pltpu.make_async_copy(lnb_hbm, lnb_v, sem_small),
        pltpu.make_async_copy(w2_hbm, w2_v, sem_small),
        pltpu.make_async_copy(b2_hbm, b2_v, sem_small),
    ]
    for cp in small:
        cp.start()
    for cp in small:
        cp.wait()
    cp_w1.wait()

    h = _dotT(z_v[...], w1_v[:, 0:4096])
    h = h + _dotT(tf_v[...], w1_v[:, 4096:4097])
    h = h + _dotT(tr_v[...], w1_v[:, 4097:4098])
    h = h + b1_v[...]
    h = jnp.maximum(h, 0.0) + a_v[...] * jnp.minimum(h, 0.0)
    mu = jnp.mean(h, axis=1, keepdims=True)
    d = h - mu
    var = jnp.mean(d * d, axis=1, keepdims=True)
    hn = d * jax.lax.rsqrt(var + 1e-5) * lnw_v[...] + lnb_v[...]
    logits = _dotT(hn, w2_v[...]) + b2_v[...]
    m = jnp.max(logits, axis=1, keepdims=True)
    e = jnp.exp(logits - m)
    s = jnp.sum(e, axis=1, keepdims=True)
    out_ref[...] = e / s


@jax.jit
def _gate(z, tf, tr, W1, b1, a, lnw, lnb, W2, b2):
    hbm = pl.BlockSpec(memory_space=pltpu.MemorySpace.HBM)
    return pl.pallas_call(
        _gate_body,
        out_shape=jax.ShapeDtypeStruct((1, 64), jnp.float32),
        in_specs=[hbm] * 10,
        out_specs=pl.BlockSpec(memory_space=pltpu.MemorySpace.VMEM),
        scratch_shapes=[
            pltpu.VMEM((128, 4098), jnp.float32),
            pltpu.VMEM((1, 4096), jnp.float32),
            pltpu.VMEM((1, 1), jnp.float32),
            pltpu.VMEM((1, 1), jnp.float32),
            pltpu.VMEM((1, 128), jnp.float32),
            pltpu.VMEM((1, 1), jnp.float32),
            pltpu.VMEM((1, 128), jnp.float32),
            pltpu.VMEM((1, 128), jnp.float32),
            pltpu.VMEM((64, 128), jnp.float32),
            pltpu.VMEM((1, 64), jnp.float32),
            pltpu.SemaphoreType.DMA,
            pltpu.SemaphoreType.DMA,
        ],
    )(z, tf, tr, W1, b1, a, lnw, lnb, W2, b2)


def kernel(z_refined, trust_form, trust_role, W1, b1, prelu_a, ln_w, ln_b, W2, b2):
    return _gate(
        z_refined,
        trust_form.reshape(1, 1),
        trust_role.reshape(1, 1),
        W1,
        b1.reshape(1, 128),
        prelu_a.reshape(1, 1),
        ln_w.reshape(1, 128),
        ln_b.reshape(1, 128),
        W2,
        b2.reshape(1, 64),
    )
